# baseline (speedup 1.0000x reference)
import functools

import jax
import jax.numpy as jnp
from jax import lax
from jax.experimental import pallas as pl
from jax.experimental.pallas import tpu as pltpu
from jax.experimental.pallas import tpu_sc as plsc

L = 16
NCORES = 2
NSUB = 16
NW = NCORES * NSUB


def _oe_merge_net(lo, n, r, out):
    step = r * 2
    if step < n:
        _oe_merge_net(lo, n, step, out)
        _oe_merge_net(lo + r, n, step, out)
        for i in range(lo + r, lo + n - r, step):
            out.append((i, i + r))
    else:
        out.append((lo, lo + r))


def _oe_sort_net(lo, n, out):
    if n > 1:
        m = n // 2
        _oe_sort_net(lo, m, out)
        _oe_sort_net(lo + m, m, out)
        _oe_merge_net(lo, n, 1, out)


_SORT16 = []
_oe_sort_net(0, 16, _SORT16)
_MERGE32 = []
_oe_merge_net(0, 32, 1, _MERGE32)
_MERGE24 = [c for c in _MERGE32 if c[0] < 24 and c[1] < 24]


def _merge2(a, b):
    br = lax.rev(b, (0,))
    lo = jnp.minimum(a, br)
    hi = jnp.maximum(a, br)
    return jnp.sort(lo), jnp.sort(hi)


def _fine_samples_sc(origin, direction, z_vals, weights, u):
    B, Nc = z_vals.shape
    S = u.shape[1]
    N = Nc + S
    C = 32
    rays_per_w = B // NW
    nchunks = rays_per_w // C

    mesh = plsc.VectorSubcoreMesh(core_axis_name="c", subcore_axis_name="s")

    @functools.partial(
        pl.kernel,
        out_type=[
            jax.ShapeDtypeStruct((B, N * 3), jnp.float32),
            jax.ShapeDtypeStruct((B, N), jnp.float32),
        ],
        mesh=mesh,
        compiler_params=pltpu.CompilerParams(needs_layout_passes=False),
        scratch_types=[
            pltpu.VMEM((C, 3), jnp.float32),
            pltpu.VMEM((C, 3), jnp.float32),
            pltpu.VMEM((C, Nc), jnp.float32),
            pltpu.VMEM((C, Nc), jnp.float32),
            pltpu.VMEM((C, S), jnp.float32),
            pltpu.VMEM((256,), jnp.float32),
            pltpu.VMEM((256,), jnp.float32),
            pltpu.VMEM((256,), jnp.float32),
            pltpu.VMEM((256,), jnp.float32),
            pltpu.VMEM((C, N), jnp.float32),
            pltpu.VMEM((C, N * 3), jnp.float32),
        ],
    )
    def launch(o_hbm, d_hbm, z_hbm, w_hbm, u_hbm, pts_hbm, zall_hbm,
               o2, d2, z2, w2, u2, cdfb, binsb, cdfb2, binsb2, zallb, ptsb):
        wid = lax.axis_index("s") * NCORES + lax.axis_index("c")
        iota = lax.iota(jnp.int32, L)
        inf16 = jnp.full((L,), 3e38, jnp.float32)

        def chunk_body(ci, carry):
            base = wid * rays_per_w + ci * C
            pltpu.sync_copy(o_hbm.at[pl.ds(base, C)], o2)
            pltpu.sync_copy(d_hbm.at[pl.ds(base, C)], d2)
            pltpu.sync_copy(z_hbm.at[pl.ds(base, C)], z2)
            pltpu.sync_copy(w_hbm.at[pl.ds(base, C)], w2)
            pltpu.sync_copy(u_hbm.at[pl.ds(base, C)], u2)

            def process_ray(r, cdfb, binsb):
                rv = jnp.full((L,), r, jnp.int32)

                plsc.store_scatter(cdfb, [iota], jnp.zeros((L,), jnp.float32))
                wsum = jnp.float32(0.0)
                wvs = []
                for j in range(8):
                    idx = jnp.minimum(iota + (1 + L * j), Nc - 1)
                    wv = plsc.load_gather(w2, [rv, idx])
                    if j == 7:
                        wv = jnp.where(iota < 14, wv + 1e-5, 0.0)
                    else:
                        wv = wv + 1e-5
                    wvs.append(wv)
                    wsum = wsum + jnp.sum(wv)
                rcp = 1.0 / lax.broadcast_in_dim(wsum, (L,), ())
                run = jnp.float32(0.0)
                for j in range(8):
                    cs = plsc.cumsum(wvs[j] * rcp) + run
                    run = jnp.max(cs)
                    if j == 7:
                        cs = jnp.where(iota >= 14, 3e38, cs)
                    plsc.store_scatter(cdfb, [iota + (1 + L * j)], cs)

                for j in range(8):
                    za = z2[r, pl.ds(L * j, L)]
                    zb = plsc.load_gather(z2, [rv, jnp.minimum(iota + L * j + 1, Nc - 1)])
                    plsc.store_scatter(binsb, [iota + L * j], 0.5 * (za + zb))

                ub = []
                for k in range(16):
                    ub.append(jnp.sort(u2[r, pl.ds(L * k, L)]))
                for (a, b) in _SORT16:
                    ub[a], ub[b] = _merge2(ub[a], ub[b])

                sb = []
                for k in range(16):
                    uv = ub[k]
                    pos = jnp.zeros((L,), jnp.int32)
                    for step in (64, 32, 16, 8, 4, 2, 1):
                        cand = pos + step
                        c = plsc.load_gather(cdfb, [cand])
                        pos = jnp.where(c <= uv, cand, pos)
                    above = jnp.minimum(pos + 1, 126)
                    cb = plsc.load_gather(cdfb, [pos])
                    ca = plsc.load_gather(cdfb, [above])
                    bb = plsc.load_gather(binsb, [pos])
                    ba = plsc.load_gather(binsb, [above])
                    denom = ca - cb
                    denom = jnp.where(denom < 1e-5, 1.0, denom)
                    t = (uv - cb) / denom
                    sb.append(bb + t * (ba - bb))

                blocks = sb
                for j in range(8):
                    blocks.append(z2[r, pl.ds(L * j, L)])
                for (a, b) in _MERGE24:
                    blocks[a], blocks[b] = _merge2(blocks[a], blocks[b])

                zero16 = jnp.zeros((L,), jnp.int32)
                ox = plsc.load_gather(o2, [rv, zero16])
                oy = plsc.load_gather(o2, [rv, zero16 + 1])
                oz = plsc.load_gather(o2, [rv, zero16 + 2])
                dx = plsc.load_gather(d2, [rv, zero16])
                dy = plsc.load_gather(d2, [rv, zero16 + 1])
                dz = plsc.load_gather(d2, [rv, zero16 + 2])
                for k in range(24):
                    m = blocks[k]
                    zallb[r, pl.ds(L * k, L)] = m
                    pidx = (iota + L * k) * 3
                    plsc.store_scatter(ptsb, [rv, pidx], ox + dx * m)
                    plsc.store_scatter(ptsb, [rv, pidx + 1], oy + dy * m)
                    plsc.store_scatter(ptsb, [rv, pidx + 2], oz + dz * m)

            def ray_body(i, carry2):
                process_ray(2 * i, cdfb, binsb)
                process_ray(2 * i + 1, cdfb2, binsb2)
                return carry2

            lax.fori_loop(0, C // 2, ray_body, 0)
            pltpu.sync_copy(zallb, zall_hbm.at[pl.ds(base, C)])
            pltpu.sync_copy(ptsb, pts_hbm.at[pl.ds(base, C)])
            return carry

        lax.fori_loop(0, nchunks, chunk_body, 0)

    return launch(origin, direction, z_vals, weights, u)


def kernel(origin_input, direction_input, z_vals, viewdirs, weights, u):
    B, Nc = z_vals.shape
    N = Nc + u.shape[1]
    pts_flat, z_all = _fine_samples_sc(origin_input, direction_input,
                                       z_vals, weights, u)
    return (pts_flat.reshape(B, N, 3), viewdirs, z_all)


# speedup vs baseline: 1.6647x; 1.6647x over previous
import functools

import jax
import jax.numpy as jnp
from jax import lax
from jax.experimental import pallas as pl
from jax.experimental.pallas import tpu as pltpu
from jax.experimental.pallas import tpu_sc as plsc

L = 16
NCORES = 2
NSUB = 16
NW = NCORES * NSUB


def _oe_merge_net(lo, n, r, out):
    step = r * 2
    if step < n:
        _oe_merge_net(lo, n, step, out)
        _oe_merge_net(lo + r, n, step, out)
        for i in range(lo + r, lo + n - r, step):
            out.append((i, i + r))
    else:
        out.append((lo, lo + r))


def _oe_sort_net(lo, n, out):
    if n > 1:
        m = n // 2
        _oe_sort_net(lo, m, out)
        _oe_sort_net(lo + m, m, out)
        _oe_merge_net(lo, n, 1, out)


_SORT16 = []
_oe_sort_net(0, 16, _SORT16)
_MERGE32 = []
_oe_merge_net(0, 32, 1, _MERGE32)
_MERGE24 = [c for c in _MERGE32 if c[0] < 24 and c[1] < 24]


def _merge2(a, b):
    br = lax.rev(b, (0,))
    lo = jnp.minimum(a, br)
    hi = jnp.maximum(a, br)
    return jnp.sort(lo), jnp.sort(hi)


def _fine_samples_sc(origin, direction, z_vals, weights, u):
    B, Nc = z_vals.shape
    S = u.shape[1]
    N = Nc + S
    C = 16
    rays_per_w = B // NW
    nchunks = rays_per_w // C
    H = nchunks // 2

    mesh = plsc.VectorSubcoreMesh(core_axis_name="c", subcore_axis_name="s")

    def _set():
        return [
            pltpu.VMEM((C, 3), jnp.float32),
            pltpu.VMEM((C, 3), jnp.float32),
            pltpu.VMEM((C, Nc), jnp.float32),
            pltpu.VMEM((C, Nc), jnp.float32),
            pltpu.VMEM((C, S), jnp.float32),
            pltpu.VMEM((C, N), jnp.float32),
            pltpu.VMEM((C, N * 3), jnp.float32),
        ]

    @functools.partial(
        pl.kernel,
        out_type=[
            jax.ShapeDtypeStruct((B, N * 3), jnp.float32),
            jax.ShapeDtypeStruct((B, N), jnp.float32),
        ],
        mesh=mesh,
        compiler_params=pltpu.CompilerParams(needs_layout_passes=False),
        scratch_types=_set() + _set() + [
            pltpu.VMEM((256,), jnp.float32),
            pltpu.VMEM((256,), jnp.float32),
            pltpu.SemaphoreType.DMA,
            pltpu.SemaphoreType.DMA,
            pltpu.SemaphoreType.DMA,
            pltpu.SemaphoreType.DMA,
        ],
    )
    def launch(o_hbm, d_hbm, z_hbm, w_hbm, u_hbm, pts_hbm, zall_hbm, *scr):
        set0 = scr[0:7]
        set1 = scr[7:14]
        cdfb, binsb, isem0, isem1, osem0, osem1 = scr[14:20]
        wid = lax.axis_index("s") * NCORES + lax.axis_index("c")
        iota = lax.iota(jnp.int32, L)

        def start_in(bufs, sem, base):
            o2, d2, z2, w2, u2, _, _ = bufs
            pltpu.async_copy(o_hbm.at[pl.ds(base, C)], o2, sem)
            pltpu.async_copy(d_hbm.at[pl.ds(base, C)], d2, sem)
            pltpu.async_copy(z_hbm.at[pl.ds(base, C)], z2, sem)
            pltpu.async_copy(w_hbm.at[pl.ds(base, C)], w2, sem)
            pltpu.async_copy(u_hbm.at[pl.ds(base, C)], u2, sem)

        def wait_in(bufs, sem):
            o2, d2, z2, w2, u2, _, _ = bufs
            pltpu.make_async_copy(o_hbm.at[pl.ds(0, C)], o2, sem).wait()
            pltpu.make_async_copy(d_hbm.at[pl.ds(0, C)], d2, sem).wait()
            pltpu.make_async_copy(z_hbm.at[pl.ds(0, C)], z2, sem).wait()
            pltpu.make_async_copy(w_hbm.at[pl.ds(0, C)], w2, sem).wait()
            pltpu.make_async_copy(u_hbm.at[pl.ds(0, C)], u2, sem).wait()

        def start_out(bufs, sem, base):
            zallb, ptsb = bufs[5], bufs[6]
            pltpu.async_copy(zallb, zall_hbm.at[pl.ds(base, C)], sem)
            pltpu.async_copy(ptsb, pts_hbm.at[pl.ds(base, C)], sem)

        def wait_out(bufs, sem):
            zallb, ptsb = bufs[5], bufs[6]
            pltpu.make_async_copy(zallb, zall_hbm.at[pl.ds(0, C)], sem).wait()
            pltpu.make_async_copy(ptsb, pts_hbm.at[pl.ds(0, C)], sem).wait()

        def compute_chunk(bufs):
            o2, d2, z2, w2, u2, zallb, ptsb = bufs

            def ray_body(r, carry2):
                rv = jnp.full((L,), r, jnp.int32)

                plsc.store_scatter(cdfb, [iota], jnp.zeros((L,), jnp.float32))
                wsum = jnp.float32(0.0)
                wvs = []
                for j in range(8):
                    idx = jnp.minimum(iota + (1 + L * j), Nc - 1)
                    wv = plsc.load_gather(w2, [rv, idx])
                    if j == 7:
                        wv = jnp.where(iota < 14, wv + 1e-5, 0.0)
                    else:
                        wv = wv + 1e-5
                    wvs.append(wv)
                    wsum = wsum + jnp.sum(wv)
                rcp = 1.0 / lax.broadcast_in_dim(wsum, (L,), ())
                run = jnp.float32(0.0)
                for j in range(8):
                    cs = plsc.cumsum(wvs[j] * rcp) + run
                    run = jnp.max(cs)
                    if j == 7:
                        cs = jnp.where(iota >= 14, 3e38, cs)
                    plsc.store_scatter(cdfb, [iota + (1 + L * j)], cs)

                for j in range(8):
                    za = z2[r, pl.ds(L * j, L)]
                    zb = plsc.load_gather(
                        z2, [rv, jnp.minimum(iota + L * j + 1, Nc - 1)])
                    plsc.store_scatter(binsb, [iota + L * j], 0.5 * (za + zb))

                ub = []
                for k in range(16):
                    ub.append(jnp.sort(u2[r, pl.ds(L * k, L)]))
                for (a, b) in _SORT16:
                    ub[a], ub[b] = _merge2(ub[a], ub[b])

                sb = []
                for k in range(16):
                    uv = ub[k]
                    pos = jnp.zeros((L,), jnp.int32)
                    for step in (64, 32, 16, 8, 4, 2, 1):
                        cand = pos + step
                        c = plsc.load_gather(cdfb, [cand])
                        pos = jnp.where(c <= uv, cand, pos)
                    above = jnp.minimum(pos + 1, 126)
                    cb = plsc.load_gather(cdfb, [pos])
                    ca = plsc.load_gather(cdfb, [above])
                    bb = plsc.load_gather(binsb, [pos])
                    ba = plsc.load_gather(binsb, [above])
                    denom = ca - cb
                    denom = jnp.where(denom < 1e-5, 1.0, denom)
                    t = (uv - cb) / denom
                    sb.append(bb + t * (ba - bb))

                blocks = sb
                for j in range(8):
                    blocks.append(z2[r, pl.ds(L * j, L)])
                for (a, b) in _MERGE24:
                    blocks[a], blocks[b] = _merge2(blocks[a], blocks[b])

                zero16 = jnp.zeros((L,), jnp.int32)
                ox = plsc.load_gather(o2, [rv, zero16])
                oy = plsc.load_gather(o2, [rv, zero16 + 1])
                oz = plsc.load_gather(o2, [rv, zero16 + 2])
                dx = plsc.load_gather(d2, [rv, zero16])
                dy = plsc.load_gather(d2, [rv, zero16 + 1])
                dz = plsc.load_gather(d2, [rv, zero16 + 2])
                for k in range(24):
                    m = blocks[k]
                    zallb[r, pl.ds(L * k, L)] = m
                    pidx = (iota + L * k) * 3
                    plsc.store_scatter(ptsb, [rv, pidx], ox + dx * m)
                    plsc.store_scatter(ptsb, [rv, pidx + 1], oy + dy * m)
                    plsc.store_scatter(ptsb, [rv, pidx + 2], oz + dz * m)
                return carry2

            lax.fori_loop(0, C, ray_body, 0)

        start_in(set0, isem0, wid * rays_per_w)

        def h_body(h, carry):
            base0 = wid * rays_per_w + (2 * h) * C
            base1 = base0 + C
            wait_in(set0, isem0)
            start_in(set1, isem1, base1)

            @pl.when(h > 0)
            def _():
                wait_out(set0, osem0)

            compute_chunk(set0)
            start_out(set0, osem0, base0)
            wait_in(set1, isem1)

            @pl.when(h < H - 1)
            def _():
                start_in(set0, isem0, base0 + 2 * C)

            @pl.when(h > 0)
            def _():
                wait_out(set1, osem1)

            compute_chunk(set1)
            start_out(set1, osem1, base1)
            return carry

        lax.fori_loop(0, H, h_body, 0)
        wait_out(set0, osem0)
        wait_out(set1, osem1)

    return launch(origin, direction, z_vals, weights, u)


def kernel(origin_input, direction_input, z_vals, viewdirs, weights, u):
    B, Nc = z_vals.shape
    N = Nc + u.shape[1]
    pts_flat, z_all = _fine_samples_sc(origin_input, direction_input,
                                       z_vals, weights, u)
    return (pts_flat.reshape(B, N, 3), viewdirs, z_all)


# speedup vs baseline: 2.5128x; 1.5094x over previous
import functools

import jax
import jax.numpy as jnp
from jax import lax
from jax.experimental import pallas as pl
from jax.experimental.pallas import tpu as pltpu
from jax.experimental.pallas import tpu_sc as plsc

L = 16
NCORES = 2
NSUB = 16
NW = NCORES * NSUB


def _oe_merge_net(lo, n, r, out):
    step = r * 2
    if step < n:
        _oe_merge_net(lo, n, step, out)
        _oe_merge_net(lo + r, n, step, out)
        for i in range(lo + r, lo + n - r, step):
            out.append((i, i + r))
    else:
        out.append((lo, lo + r))


def _oe_sort_net(lo, n, out):
    if n > 1:
        m = n // 2
        _oe_sort_net(lo, m, out)
        _oe_sort_net(lo + m, m, out)
        _oe_merge_net(lo, n, 1, out)


_SORT16 = []
_oe_sort_net(0, 16, _SORT16)
_MERGE32 = []
_oe_merge_net(0, 32, 1, _MERGE32)
_MERGE24 = [c for c in _MERGE32 if c[0] < 24 and c[1] < 24]


def _merge2(a, b):
    br = lax.rev(b, (0,))
    lo = jnp.minimum(a, br)
    hi = jnp.maximum(a, br)
    return jnp.sort(lo), jnp.sort(hi)


def _fine_samples_sc(origin, direction, z_vals, weights, u):
    B, Nc = z_vals.shape
    S = u.shape[1]
    N = Nc + S
    C = 16
    rays_per_w = B // NW
    nchunks = rays_per_w // C
    H = nchunks // 2

    mesh = plsc.VectorSubcoreMesh(core_axis_name="c", subcore_axis_name="s")

    def _set():
        return [
            pltpu.VMEM((C, 3), jnp.float32),
            pltpu.VMEM((C, 3), jnp.float32),
            pltpu.VMEM((C, Nc), jnp.float32),
            pltpu.VMEM((C, Nc), jnp.float32),
            pltpu.VMEM((C, S), jnp.float32),
            pltpu.VMEM((C, N), jnp.float32),
            pltpu.VMEM((C, N), jnp.float32),
            pltpu.VMEM((C, N), jnp.float32),
            pltpu.VMEM((C, N), jnp.float32),
        ]

    @functools.partial(
        pl.kernel,
        out_type=[
            jax.ShapeDtypeStruct((3, B, N), jnp.float32),
            jax.ShapeDtypeStruct((B, N), jnp.float32),
        ],
        mesh=mesh,
        compiler_params=pltpu.CompilerParams(needs_layout_passes=False),
        scratch_types=_set() + _set() + [
            pltpu.VMEM((256,), jnp.float32),
            pltpu.VMEM((256,), jnp.float32),
            pltpu.SemaphoreType.DMA,
            pltpu.SemaphoreType.DMA,
            pltpu.SemaphoreType.DMA,
            pltpu.SemaphoreType.DMA,
        ],
    )
    def launch(o_hbm, d_hbm, z_hbm, w_hbm, u_hbm, pts_hbm, zall_hbm, *scr):
        set0 = scr[0:9]
        set1 = scr[9:18]
        cdfb, binsb, isem0, isem1, osem0, osem1 = scr[18:24]
        wid = lax.axis_index("s") * NCORES + lax.axis_index("c")
        iota = lax.iota(jnp.int32, L)

        def start_in(bufs, sem, base):
            o2, d2, z2, w2, u2 = bufs[:5]
            pltpu.async_copy(o_hbm.at[pl.ds(base, C)], o2, sem)
            pltpu.async_copy(d_hbm.at[pl.ds(base, C)], d2, sem)
            pltpu.async_copy(z_hbm.at[pl.ds(base, C)], z2, sem)
            pltpu.async_copy(w_hbm.at[pl.ds(base, C)], w2, sem)
            pltpu.async_copy(u_hbm.at[pl.ds(base, C)], u2, sem)

        def wait_in(bufs, sem):
            o2, d2, z2, w2, u2 = bufs[:5]
            pltpu.make_async_copy(o_hbm.at[pl.ds(0, C)], o2, sem).wait()
            pltpu.make_async_copy(d_hbm.at[pl.ds(0, C)], d2, sem).wait()
            pltpu.make_async_copy(z_hbm.at[pl.ds(0, C)], z2, sem).wait()
            pltpu.make_async_copy(w_hbm.at[pl.ds(0, C)], w2, sem).wait()
            pltpu.make_async_copy(u_hbm.at[pl.ds(0, C)], u2, sem).wait()

        def start_out(bufs, sem, base):
            zallb, xb, yb, zb = bufs[5:9]
            pltpu.async_copy(zallb, zall_hbm.at[pl.ds(base, C)], sem)
            pltpu.async_copy(xb, pts_hbm.at[0, pl.ds(base, C)], sem)
            pltpu.async_copy(yb, pts_hbm.at[1, pl.ds(base, C)], sem)
            pltpu.async_copy(zb, pts_hbm.at[2, pl.ds(base, C)], sem)

        def wait_out(bufs, sem):
            zallb, xb, yb, zb = bufs[5:9]
            pltpu.make_async_copy(zallb, zall_hbm.at[pl.ds(0, C)], sem).wait()
            pltpu.make_async_copy(xb, pts_hbm.at[0, pl.ds(0, C)], sem).wait()
            pltpu.make_async_copy(yb, pts_hbm.at[1, pl.ds(0, C)], sem).wait()
            pltpu.make_async_copy(zb, pts_hbm.at[2, pl.ds(0, C)], sem).wait()

        def compute_chunk(bufs):
            o2, d2, z2, w2, u2, zallb, xb, yb, zb = bufs

            def ray_body(r, carry2):
                rv = jnp.full((L,), r, jnp.int32)

                plsc.store_scatter(cdfb, [iota], jnp.zeros((L,), jnp.float32))
                wsum = jnp.float32(0.0)
                wvs = []
                for j in range(8):
                    idx = jnp.minimum(iota + (1 + L * j), Nc - 1)
                    wv = plsc.load_gather(w2, [rv, idx])
                    if j == 7:
                        wv = jnp.where(iota < 14, wv + 1e-5, 0.0)
                    else:
                        wv = wv + 1e-5
                    wvs.append(wv)
                    wsum = wsum + jnp.sum(wv)
                rcp = 1.0 / lax.broadcast_in_dim(wsum, (L,), ())
                run = jnp.float32(0.0)
                for j in range(8):
                    cs = plsc.cumsum(wvs[j] * rcp) + run
                    run = jnp.max(cs)
                    if j == 7:
                        cs = jnp.where(iota >= 14, 3e38, cs)
                    plsc.store_scatter(cdfb, [iota + (1 + L * j)], cs)

                for j in range(8):
                    zlo = z2[r, pl.ds(L * j, L)]
                    zhi = plsc.load_gather(
                        z2, [rv, jnp.minimum(iota + L * j + 1, Nc - 1)])
                    plsc.store_scatter(binsb, [iota + L * j], 0.5 * (zlo + zhi))

                ub = []
                for k in range(16):
                    ub.append(jnp.sort(u2[r, pl.ds(L * k, L)]))
                for (a, b) in _SORT16:
                    ub[a], ub[b] = _merge2(ub[a], ub[b])

                sb = []
                for k in range(16):
                    uv = ub[k]
                    pos = jnp.zeros((L,), jnp.int32)
                    for step in (64, 32, 16, 8, 4, 2, 1):
                        cand = pos + step
                        c = plsc.load_gather(cdfb, [cand])
                        pos = jnp.where(c <= uv, cand, pos)
                    above = jnp.minimum(pos + 1, 126)
                    cb = plsc.load_gather(cdfb, [pos])
                    ca = plsc.load_gather(cdfb, [above])
                    bb = plsc.load_gather(binsb, [pos])
                    ba = plsc.load_gather(binsb, [above])
                    denom = ca - cb
                    denom = jnp.where(denom < 1e-5, 1.0, denom)
                    t = (uv - cb) / denom
                    sb.append(bb + t * (ba - bb))

                blocks = sb
                for j in range(8):
                    blocks.append(z2[r, pl.ds(L * j, L)])
                for (a, b) in _MERGE24:
                    blocks[a], blocks[b] = _merge2(blocks[a], blocks[b])

                zero16 = jnp.zeros((L,), jnp.int32)
                ox = plsc.load_gather(o2, [rv, zero16])
                oy = plsc.load_gather(o2, [rv, zero16 + 1])
                oz = plsc.load_gather(o2, [rv, zero16 + 2])
                dx = plsc.load_gather(d2, [rv, zero16])
                dy = plsc.load_gather(d2, [rv, zero16 + 1])
                dz = plsc.load_gather(d2, [rv, zero16 + 2])
                for k in range(24):
                    m = blocks[k]
                    sl = pl.ds(L * k, L)
                    zallb[r, sl] = m
                    xb[r, sl] = ox + dx * m
                    yb[r, sl] = oy + dy * m
                    zb[r, sl] = oz + dz * m
                return carry2

            lax.fori_loop(0, C, ray_body, 0)

        start_in(set0, isem0, wid * rays_per_w)

        def h_body(h, carry):
            base0 = wid * rays_per_w + (2 * h) * C
            base1 = base0 + C
            wait_in(set0, isem0)
            start_in(set1, isem1, base1)

            @pl.when(h > 0)
            def _():
                wait_out(set0, osem0)

            compute_chunk(set0)
            start_out(set0, osem0, base0)
            wait_in(set1, isem1)

            @pl.when(h < H - 1)
            def _():
                start_in(set0, isem0, base0 + 2 * C)

            @pl.when(h > 0)
            def _():
                wait_out(set1, osem1)

            compute_chunk(set1)
            start_out(set1, osem1, base1)
            return carry

        lax.fori_loop(0, H, h_body, 0)
        wait_out(set0, osem0)
        wait_out(set1, osem1)

    return launch(origin, direction, z_vals, weights, u)


def kernel(origin_input, direction_input, z_vals, viewdirs, weights, u):
    pts_planar, z_all = _fine_samples_sc(origin_input, direction_input,
                                         z_vals, weights, u)
    return (jnp.transpose(pts_planar, (1, 2, 0)), viewdirs, z_all)


# speedup vs baseline: 2.6313x; 1.0472x over previous
import functools

import jax
import jax.numpy as jnp
from jax import lax
from jax.experimental import pallas as pl
from jax.experimental.pallas import tpu as pltpu
from jax.experimental.pallas import tpu_sc as plsc

L = 16
NCORES = 2
NSUB = 16
NW = NCORES * NSUB


def _oe_merge_net(lo, n, r, out):
    step = r * 2
    if step < n:
        _oe_merge_net(lo, n, step, out)
        _oe_merge_net(lo + r, n, step, out)
        for i in range(lo + r, lo + n - r, step):
            out.append((i, i + r))
    else:
        out.append((lo, lo + r))


def _oe_sort_net(lo, n, out):
    if n > 1:
        m = n // 2
        _oe_sort_net(lo, m, out)
        _oe_sort_net(lo + m, m, out)
        _oe_merge_net(lo, n, 1, out)


_SORT16 = []
_oe_sort_net(0, 16, _SORT16)
_MERGE32 = []
_oe_merge_net(0, 32, 1, _MERGE32)
_MERGE24 = [c for c in _MERGE32 if c[0] < 24 and c[1] < 24]


def _merge2(a, b):
    br = lax.rev(b, (0,))
    lo = jnp.minimum(a, br)
    hi = jnp.maximum(a, br)
    return jnp.sort(lo), jnp.sort(hi)


def _fine_samples_sc(origin, direction, z_vals, weights, u):
    B, Nc = z_vals.shape
    S = u.shape[1]
    N = Nc + S
    C = 16
    rays_per_w = B // NW
    nchunks = rays_per_w // C
    H = nchunks // 2

    mesh = plsc.VectorSubcoreMesh(core_axis_name="c", subcore_axis_name="s")

    def _set():
        return [
            pltpu.VMEM((C, 3), jnp.float32),
            pltpu.VMEM((C, 3), jnp.float32),
            pltpu.VMEM((C, Nc), jnp.float32),
            pltpu.VMEM((C, Nc), jnp.float32),
            pltpu.VMEM((C, S), jnp.float32),
            pltpu.VMEM((C, N), jnp.float32),
            pltpu.VMEM((C, N), jnp.float32),
            pltpu.VMEM((C, N), jnp.float32),
            pltpu.VMEM((C, N), jnp.float32),
        ]

    @functools.partial(
        pl.kernel,
        out_type=[
            jax.ShapeDtypeStruct((3, B, N), jnp.float32),
            jax.ShapeDtypeStruct((B, N), jnp.float32),
        ],
        mesh=mesh,
        compiler_params=pltpu.CompilerParams(needs_layout_passes=False),
        scratch_types=_set() + _set() + [
            pltpu.VMEM((256,), jnp.float32),
            pltpu.VMEM((256,), jnp.float32),
            pltpu.SemaphoreType.DMA,
            pltpu.SemaphoreType.DMA,
            pltpu.SemaphoreType.DMA,
            pltpu.SemaphoreType.DMA,
        ],
    )
    def launch(o_hbm, d_hbm, z_hbm, w_hbm, u_hbm, pts_hbm, zall_hbm, *scr):
        set0 = scr[0:9]
        set1 = scr[9:18]
        cdfb, binsb, isem0, isem1, osem0, osem1 = scr[18:24]
        wid = lax.axis_index("s") * NCORES + lax.axis_index("c")
        iota = lax.iota(jnp.int32, L)

        def start_in(bufs, sem, base):
            o2, d2, z2, w2, u2 = bufs[:5]
            pltpu.async_copy(o_hbm.at[pl.ds(base, C)], o2, sem)
            pltpu.async_copy(d_hbm.at[pl.ds(base, C)], d2, sem)
            pltpu.async_copy(z_hbm.at[pl.ds(base, C)], z2, sem)
            pltpu.async_copy(w_hbm.at[pl.ds(base, C)], w2, sem)
            pltpu.async_copy(u_hbm.at[pl.ds(base, C)], u2, sem)

        def wait_in(bufs, sem):
            o2, d2, z2, w2, u2 = bufs[:5]
            pltpu.make_async_copy(o_hbm.at[pl.ds(0, C)], o2, sem).wait()
            pltpu.make_async_copy(d_hbm.at[pl.ds(0, C)], d2, sem).wait()
            pltpu.make_async_copy(z_hbm.at[pl.ds(0, C)], z2, sem).wait()
            pltpu.make_async_copy(w_hbm.at[pl.ds(0, C)], w2, sem).wait()
            pltpu.make_async_copy(u_hbm.at[pl.ds(0, C)], u2, sem).wait()

        def start_out(bufs, sem, base):
            zallb, xb, yb, zb = bufs[5:9]
            pltpu.async_copy(zallb, zall_hbm.at[pl.ds(base, C)], sem)
            pltpu.async_copy(xb, pts_hbm.at[0, pl.ds(base, C)], sem)
            pltpu.async_copy(yb, pts_hbm.at[1, pl.ds(base, C)], sem)
            pltpu.async_copy(zb, pts_hbm.at[2, pl.ds(base, C)], sem)

        def wait_out(bufs, sem):
            zallb, xb, yb, zb = bufs[5:9]
            pltpu.make_async_copy(zallb, zall_hbm.at[pl.ds(0, C)], sem).wait()
            pltpu.make_async_copy(xb, pts_hbm.at[0, pl.ds(0, C)], sem).wait()
            pltpu.make_async_copy(yb, pts_hbm.at[1, pl.ds(0, C)], sem).wait()
            pltpu.make_async_copy(zb, pts_hbm.at[2, pl.ds(0, C)], sem).wait()

        def compute_chunk(bufs):
            o2, d2, z2, w2, u2, zallb, xb, yb, zb = bufs

            def ray_body(r, carry2):
                rv = jnp.full((L,), r, jnp.int32)

                plsc.store_scatter(cdfb, [iota], jnp.zeros((L,), jnp.float32))
                wvs = []
                wacc = None
                for j in range(8):
                    idx = jnp.minimum(iota + (1 + L * j), Nc - 1)
                    wv = plsc.load_gather(w2, [rv, idx])
                    if j == 7:
                        wv = jnp.where(iota < 14, wv + 1e-5, 0.0)
                    else:
                        wv = wv + 1e-5
                    wvs.append(wv)
                    wacc = wv if wacc is None else wacc + wv
                wsum = jnp.sum(wacc)
                rcp = 1.0 / lax.broadcast_in_dim(wsum, (L,), ())
                run = jnp.float32(0.0)
                for j in range(8):
                    cs = plsc.cumsum(wvs[j] * rcp) + run
                    run = cs[15]
                    if j == 7:
                        cs = jnp.where(iota >= 14, 3e38, cs)
                    plsc.store_scatter(cdfb, [iota + (1 + L * j)], cs)

                for j in range(8):
                    zlo = z2[r, pl.ds(L * j, L)]
                    zhi = plsc.load_gather(
                        z2, [rv, jnp.minimum(iota + L * j + 1, Nc - 1)])
                    binsb[pl.ds(L * j, L)] = 0.5 * (zlo + zhi)

                ub = []
                for k in range(16):
                    ub.append(jnp.sort(u2[r, pl.ds(L * k, L)]))
                for (a, b) in _SORT16:
                    ub[a], ub[b] = _merge2(ub[a], ub[b])

                sb = []
                for k in range(16):
                    uv = ub[k]
                    pos = jnp.zeros((L,), jnp.int32)
                    for step in (64, 32, 16, 8, 4, 2, 1):
                        cand = pos + step
                        c = plsc.load_gather(cdfb, [cand])
                        pos = jnp.where(c <= uv, cand, pos)
                    above = jnp.minimum(pos + 1, 126)
                    cb = plsc.load_gather(cdfb, [pos])
                    ca = plsc.load_gather(cdfb, [above])
                    bb = plsc.load_gather(binsb, [pos])
                    ba = plsc.load_gather(binsb, [above])
                    denom = ca - cb
                    denom = jnp.where(denom < 1e-5, 1.0, denom)
                    t = (uv - cb) / denom
                    sb.append(bb + t * (ba - bb))

                blocks = sb
                for j in range(8):
                    blocks.append(z2[r, pl.ds(L * j, L)])
                for (a, b) in _MERGE24:
                    blocks[a], blocks[b] = _merge2(blocks[a], blocks[b])

                zero16 = jnp.zeros((L,), jnp.int32)
                ox = plsc.load_gather(o2, [rv, zero16])
                oy = plsc.load_gather(o2, [rv, zero16 + 1])
                oz = plsc.load_gather(o2, [rv, zero16 + 2])
                dx = plsc.load_gather(d2, [rv, zero16])
                dy = plsc.load_gather(d2, [rv, zero16 + 1])
                dz = plsc.load_gather(d2, [rv, zero16 + 2])
                for k in range(24):
                    m = blocks[k]
                    sl = pl.ds(L * k, L)
                    zallb[r, sl] = m
                    xb[r, sl] = ox + dx * m
                    yb[r, sl] = oy + dy * m
                    zb[r, sl] = oz + dz * m
                return carry2

            lax.fori_loop(0, C, ray_body, 0)

        start_in(set0, isem0, wid * rays_per_w)

        def h_body(h, carry):
            base0 = wid * rays_per_w + (2 * h) * C
            base1 = base0 + C
            wait_in(set0, isem0)
            start_in(set1, isem1, base1)

            @pl.when(h > 0)
            def _():
                wait_out(set0, osem0)

            compute_chunk(set0)
            start_out(set0, osem0, base0)
            wait_in(set1, isem1)

            @pl.when(h < H - 1)
            def _():
                start_in(set0, isem0, base0 + 2 * C)

            @pl.when(h > 0)
            def _():
                wait_out(set1, osem1)

            compute_chunk(set1)
            start_out(set1, osem1, base1)
            return carry

        lax.fori_loop(0, H, h_body, 0)
        wait_out(set0, osem0)
        wait_out(set1, osem1)

    return launch(origin, direction, z_vals, weights, u)


def kernel(origin_input, direction_input, z_vals, viewdirs, weights, u):
    pts_planar, z_all = _fine_samples_sc(origin_input, direction_input,
                                         z_vals, weights, u)
    return (jnp.transpose(pts_planar, (1, 2, 0)), viewdirs, z_all)
